# Initial kernel scaffold; baseline (speedup 1.0000x reference)
import jax
import jax.numpy as jnp
from jax.experimental import pallas as pl


def kernel(struct_adj, raw_feat, edge_index, struct_assign, fnc_assign, W_fg, b_fg, W_sg, b_sg, W_gat, a_gat, W_lc, b_lc, W_ls, b_ls):
    raise NotImplementedError("write your pallas kernel here")


# speedup vs baseline: 4.1659x; 4.1659x over previous
import functools

import jax
import jax.numpy as jnp
from jax import lax
from jax.experimental import pallas as pl
from jax.experimental.pallas import tpu as pltpu
from jax.experimental.pallas import tpu_sc as plsc

N = 10000
E = 320000
D = 128
KS = 512
KF = 128
ALPHA = 0.2

DP = D
NC = 2
NS = 16
NW = NC * NS
EPW = E // NW
CH = 80
NCHUNK = EPW // CH
N_PAD = 10240
ROWS_PT = N_PAD // NS

_NEG_BIG = -3.0e38


def _semb_kernel(sa_ref, rf_ref, o_ref):
    o_ref[...] = lax.dot_general(sa_ref[...], rf_ref[...], (((0,), (0,)), ((), ())))


def _dot00_kernel(a_ref, b_ref, o_ref):
    o_ref[...] = lax.dot_general(a_ref[...], b_ref[...], (((0,), (0,)), ((), ())))


def _dot11_kernel(a_ref, o_ref):
    x = a_ref[...]
    o_ref[...] = lax.dot_general(x, x, (((1,), (1,)), ((), ())))


def _dot_kernel(a_ref, b_ref, o_ref):
    o_ref[...] = jnp.dot(a_ref[...], b_ref[...])


def _dot_bias_kernel(a_ref, b_ref, c_ref, o_ref):
    o_ref[...] = jnp.dot(a_ref[...], b_ref[...]) + c_ref[...]


def _fm_kernel(fnc_ref, fe1_ref, fd_ref, se_ref, o_ref):
    fnc_message = jnp.dot(fnc_ref[...], fe1_ref[...]) / fd_ref[...]
    o_ref[...] = se_ref[...] + 0.15 * fnc_message


def _rf_kernel(sa_ref, rfeat_ref, se3_ref, o_ref):
    sm = jnp.dot(sa_ref[...], se3_ref[...])
    o_ref[...] = rfeat_ref[...] + 0.5 * sm


def _node_kernel(rf_ref, wg_ref, ap_ref, hpad_ref, s12_ref):
    h = jnp.dot(rf_ref[...], wg_ref[...])
    hpad_ref[...] = h
    s12_ref[...] = jnp.dot(h, ap_ref[...])


def _finish_kernel(p_ref, rs_ref, o_ref):
    hp = p_ref[0] + p_ref[1]
    rs = jnp.sum(rs_ref[...], axis=0)
    x = hp / (rs[:, None] + 1e-15)
    o_ref[...] = jnp.where(x > 0.0, x, jnp.exp(x) - 1.0)


def _sc_max_body(src_hbm, dst_hbm, s12_hbm, mx_hbm, eva_hbm,
                 s1_v, s2_v, src_v, dst_v, eva_v, st_v):
    cid = lax.axis_index("c")
    sid = lax.axis_index("s")
    wid = cid * NS + sid
    pltpu.sync_copy(s12_hbm.at[0], s1_v)
    pltpu.sync_copy(s12_hbm.at[1], s2_v)
    base = wid * EPW
    CB = 2000

    def chunk(c, mx):
        pltpu.sync_copy(src_hbm.at[pl.ds(base + c * CB, CB)], src_v)
        pltpu.sync_copy(dst_hbm.at[pl.ds(base + c * CB, CB)], dst_v)

        def grp(g, mx):
            sv = src_v[pl.ds(g * 16, 16)]
            dv = dst_v[pl.ds(g * 16, 16)]
            v = plsc.load_gather(s1_v, [sv]) + plsc.load_gather(s2_v, [dv])
            eva = jnp.maximum(v, ALPHA * v)
            eva_v[pl.ds(g * 16, 16)] = eva
            return jnp.maximum(mx, eva)

        mx = lax.fori_loop(0, CB // 16, grp, mx)
        pltpu.sync_copy(eva_v, eva_hbm.at[pl.ds(base + c * CB, CB)])
        return mx

    mx = lax.fori_loop(0, EPW // CB, chunk,
                       jnp.full((16,), _NEG_BIG, jnp.float32))
    st_v[...] = mx
    pltpu.sync_copy(st_v, mx_hbm.at[wid])


def _sc_acc_body(src_hbm, dst_hbm, eva_hbm, hpad_hbm, mx_hbm, part_hbm, rsp_hbm,
                 src_c, dst_c, w_v, rows_v, mx_v, rs_v, acc_sh, sem):
    cid = lax.axis_index("c")
    sid = lax.axis_index("s")
    wid = cid * NS + sid

    def zrow(i, _):
        for k in range(DP // 16):
            rows_v[i, pl.ds(k * 16, 16)] = jnp.zeros((16,), jnp.float32)
        return 0

    lax.fori_loop(0, CH, zrow, 0)
    for j in range(ROWS_PT // CH):
        pltpu.sync_copy(rows_v, acc_sh.at[pl.ds(sid * ROWS_PT + j * CH, CH)])

    def zrs(i, _):
        rs_v[pl.ds(i * 16, 16)] = jnp.zeros((16,), jnp.float32)
        return 0

    lax.fori_loop(0, N_PAD // 16, zrs, 0)
    plsc.subcore_barrier()

    pltpu.sync_copy(mx_hbm, mx_v)

    def mrow(i, m):
        return jnp.maximum(m, mx_v[i])

    mvec = lax.fori_loop(0, NW, mrow, jnp.full((16,), _NEG_BIG, jnp.float32))
    M = jnp.max(mvec)

    base = wid * EPW

    def chunk(c, _):
        pltpu.sync_copy(src_hbm.at[pl.ds(base + c * CH, CH)], src_c)
        pltpu.sync_copy(dst_hbm.at[pl.ds(base + c * CH, CH)], dst_c)
        pltpu.sync_copy(eva_hbm.at[pl.ds(base + c * CH, CH)], w_v)
        pltpu.async_copy(hpad_hbm.at[dst_c], rows_v, sem).wait()

        def grp(g, _):
            w_v[pl.ds(g * 16, 16)] = jnp.exp(w_v[pl.ds(g * 16, 16)] - M)
            return 0

        lax.fori_loop(0, CH // 16, grp, 0)

        def scale(e, _):
            eidx = jnp.full((16,), e, jnp.int32)
            w = plsc.load_gather(w_v, [eidx])
            for k in range(DP // 16):
                rows_v[e, pl.ds(k * 16, 16)] = rows_v[e, pl.ds(k * 16, 16)] * w
            si = plsc.load_gather(src_c, [eidx])
            old = plsc.load_gather(rs_v, [si])
            plsc.store_scatter(rs_v, [si], old + w)
            return 0

        lax.fori_loop(0, CH, scale, 0)
        pltpu.sync_copy(rows_v, acc_sh.at[src_c], add=True)
        return 0

    lax.fori_loop(0, NCHUNK, chunk, 0)
    pltpu.sync_copy(rs_v, rsp_hbm.at[wid])
    plsc.subcore_barrier()
    pltpu.sync_copy(acc_sh.at[pl.ds(sid * ROWS_PT, ROWS_PT)],
                    part_hbm.at[cid, pl.ds(sid * ROWS_PT, ROWS_PT)])


def kernel(struct_adj, raw_feat, edge_index, struct_assign, fnc_assign,
           W_fg, b_fg, W_sg, b_sg, W_gat, a_gat, W_lc, b_lc, W_ls, b_ls):
    BN = 1000
    eyeS = jnp.eye(KS, dtype=jnp.float32)
    eyeF = jnp.eye(KF, dtype=jnp.float32)

    sa = struct_assign / (jax.nn.relu(jnp.sum(struct_assign, 0) - 1.0) + 1.0)
    fa = fnc_assign / (jax.nn.relu(jnp.sum(fnc_assign, 0) - 1.0) + 1.0)

    def _single(body, out_shape, *args):
        return pl.pallas_call(
            body, out_shape=jax.ShapeDtypeStruct(out_shape, jnp.float32))(*args)

    struct_emb = _single(_semb_kernel, (KS, D), sa, raw_feat)
    fnc_emb0 = _single(_dot00_kernel, (KF, D), fa, struct_emb)
    logits = _single(_dot11_kernel, (KF, KF), fnc_emb0)

    fnc_adj2 = jax.nn.sigmoid(logits) + eyeF + eyeF
    deg_f = jnp.sum(jnp.abs(fnc_adj2), axis=-1)
    ds_f = deg_f ** -0.5
    norm_f = ds_f[:, None] * fnc_adj2 * ds_f[None, :]
    fdenom = (jax.nn.relu(jnp.sum(fa, 1) - 1.0) + 1.0)[:, None]

    support_f = _single(_dot_kernel, (KF, D), fnc_emb0, W_fg)
    fnc_emb1 = _single(_dot_bias_kernel, (KF, D), norm_f, support_f, b_fg)
    struct_emb2 = _single(_fm_kernel, (KS, D), fnc_assign, fnc_emb1, fdenom,
                          struct_emb)

    sadj = jax.nn.relu(struct_adj - eyeS * 10000.0) + eyeS
    adj_post = sadj + eyeS
    deg_s = jnp.sum(jnp.abs(adj_post), axis=-1)
    ds_s = deg_s ** -0.5
    norm_s = ds_s[:, None] * adj_post * ds_s[None, :]

    support_s = _single(_dot_kernel, (KS, D), struct_emb2, W_sg)
    struct_emb3 = _single(_dot_bias_kernel, (KS, D), norm_s, support_s, b_sg)

    rf = pl.pallas_call(
        _rf_kernel, grid=(N // BN,),
        in_specs=[pl.BlockSpec((BN, KS), lambda i: (i, 0)),
                  pl.BlockSpec((BN, D), lambda i: (i, 0)),
                  pl.BlockSpec((KS, D), lambda i: (0, 0))],
        out_specs=pl.BlockSpec((BN, D), lambda i: (i, 0)),
        out_shape=jax.ShapeDtypeStruct((N, D), jnp.float32),
    )(struct_assign, raw_feat, struct_emb3)

    a_pad = jnp.concatenate(
        [a_gat[0, :D][:, None], a_gat[0, D:][:, None],
         jnp.zeros((D, 6), jnp.float32)], axis=1)
    hpad, s12n = pl.pallas_call(
        _node_kernel, grid=(N // BN,),
        in_specs=[pl.BlockSpec((BN, D), lambda i: (i, 0)),
                  pl.BlockSpec((D, D), lambda i: (0, 0)),
                  pl.BlockSpec((D, 8), lambda i: (0, 0))],
        out_specs=[pl.BlockSpec((BN, DP), lambda i: (i, 0)),
                   pl.BlockSpec((BN, 8), lambda i: (i, 0))],
        out_shape=[jax.ShapeDtypeStruct((N, DP), jnp.float32),
                   jax.ShapeDtypeStruct((N, 8), jnp.float32)],
    )(rf, W_gat, a_pad)
    s12 = jnp.stack([s12n[:, 0], s12n[:, 1]], axis=0)

    src = edge_index[0]
    dst = edge_index[1]

    mesh = plsc.VectorSubcoreMesh(core_axis_name="c", subcore_axis_name="s")
    sc_params = pltpu.CompilerParams(needs_layout_passes=False)
    maxes, eva = pl.kernel(
        _sc_max_body, mesh=mesh, compiler_params=sc_params,
        out_type=[jax.ShapeDtypeStruct((NW, 16), jnp.float32),
                  jax.ShapeDtypeStruct((E,), jnp.float32)],
        scratch_types=[
            pltpu.VMEM((N,), jnp.float32),
            pltpu.VMEM((N,), jnp.float32),
            pltpu.VMEM((2000,), jnp.int32),
            pltpu.VMEM((2000,), jnp.int32),
            pltpu.VMEM((2000,), jnp.float32),
            pltpu.VMEM((16,), jnp.float32),
        ],
    )(src, dst, s12)

    part, rs_part = pl.kernel(
        _sc_acc_body, mesh=mesh, compiler_params=sc_params,
        out_type=[jax.ShapeDtypeStruct((NC, N_PAD, DP), jnp.float32),
                  jax.ShapeDtypeStruct((NW, N_PAD), jnp.float32)],
        scratch_types=[
            pltpu.VMEM((CH,), jnp.int32),
            pltpu.VMEM((CH,), jnp.int32),
            pltpu.VMEM((CH,), jnp.float32),
            pltpu.VMEM((CH, DP), jnp.float32),
            pltpu.VMEM((NW, 16), jnp.float32),
            pltpu.VMEM((N_PAD,), jnp.float32),
            pltpu.VMEM_SHARED((N_PAD, DP), jnp.float32),
            pltpu.SemaphoreType.DMA,
        ],
    )(src, dst, eva, hpad, maxes)

    BF = 1280
    out_pad = pl.pallas_call(
        _finish_kernel, grid=(N_PAD // BF,),
        in_specs=[pl.BlockSpec((NC, BF, DP), lambda i: (0, i, 0)),
                  pl.BlockSpec((NW, BF), lambda i: (0, i))],
        out_specs=pl.BlockSpec((BF, D), lambda i: (i, 0)),
        out_shape=jax.ShapeDtypeStruct((N_PAD, D), jnp.float32),
    )(part, rs_part)
    return out_pad[:N]
